# baseline (speedup 1.0000x reference)
import functools
import math

import jax
import jax.numpy as jnp
import numpy as np
from jax import lax
from jax.experimental import pallas as pl
from jax.experimental.pallas import tpu as pltpu
import jax.experimental.pallas.tpu_sc as plsc

_N = 524288
_GRID_LEVELS = 16
_GRID_LOG2_T = 19
_GRID_BASE = 16
_MAX_RES = 1024
_GRID_PLS = float(np.exp((np.log(_MAX_RES) - np.log(_GRID_BASE)) / (_GRID_LEVELS - 1)))
_PLANE_LOG2_T = 17
_PLANE_BASE = _MAX_RES // 4
_POS_DEG = 4
_P1 = np.int32(np.uint32(2654435761).astype(np.int32))
_P2 = np.int32(805459861)

_NW = 32
_PPW = _N // _NW
_C = 512
_STEPS = _C // 16
_NCH = _PPW // _C
_SUB = 4096

_GRID_RES = [int(np.floor(_GRID_BASE * (_GRID_PLS ** l))) for l in range(_GRID_LEVELS)]
_GRID_T = 1 << _GRID_LOG2_T
_PLANE_T = 1 << _PLANE_LOG2_T
_PLANE_RES = [_PLANE_BASE * (2 ** q) for q in range(4)]
_FLAT_GRID_OFF = 0
_FLAT_PLANE_OFF = _GRID_LEVELS * _GRID_T


def _emit_grid_level(l, xa, xb, xc, res):
    S = res + 1
    dense = S ** 3 <= _GRID_T
    row_off = l * _GRID_T

    def indices(j0, xav, xbv, xcv):
        rf = jnp.float32(res)
        fa = xav * rf
        fb = xbv * rf
        fc = xcv * rf
        ia = fa.astype(jnp.int32)
        ib = fb.astype(jnp.int32)
        ic = fc.astype(jnp.int32)
        lvl_off = np.int32(2 * _GRID_T * l)
        out = []
        if dense:
            base = ia + ib * np.int32(S) + ic * np.int32(S * S)
            for k in range(8):
                b0, b1, b2 = k & 1, (k >> 1) & 1, (k >> 2) & 1
                m = base + np.int32(b0 + b1 * S + b2 * S * S)
                e0 = m + m - (m & np.int32(127)) + lvl_off
                out.append((e0, e0 + np.int32(128)))
        else:
            h1 = ib * _P1
            h1p = h1 + _P1
            h2 = ic * _P2
            h2p = h2 + _P2
            iap = ia + np.int32(1)
            msk = np.int32(_GRID_T - 1)
            for k in range(8):
                b0, b1, b2 = k & 1, (k >> 1) & 1, (k >> 2) & 1
                t = (iap if b0 else ia) ^ (h1p if b1 else h1) ^ (h2p if b2 else h2)
                m = t & msk
                e0 = m + m - (m & np.int32(127)) + lvl_off
                out.append((e0, e0 + np.int32(128)))
        return out

    def weights(xav, xbv, xcv):
        rf = jnp.float32(res)
        fa = xav * rf
        fb = xbv * rf
        fc = xcv * rf
        fra = fa - fa.astype(jnp.int32).astype(jnp.float32)
        frb = fb - fb.astype(jnp.int32).astype(jnp.float32)
        frc = fc - fc.astype(jnp.int32).astype(jnp.float32)
        wa = (jnp.float32(1.0) - fra, fra)
        wb = (jnp.float32(1.0) - frb, frb)
        wc = (jnp.float32(1.0) - frc, frc)
        ws = []
        for k in range(8):
            b0, b1, b2 = k & 1, (k >> 1) & 1, (k >> 2) & 1
            ws.append(wa[b0] * wb[b1] * wc[b2])
        return ws

    return indices, weights


def _emit_plane_level(p, q):
    res = _PLANE_RES[q]
    S = res + 1
    dense = S * S <= _PLANE_T
    row_off = _FLAT_PLANE_OFF + (p * 4 + q) * _PLANE_T

    def indices(xav, xbv):
        rf = jnp.float32(res)
        fa = xav * rf
        fb = xbv * rf
        ia = fa.astype(jnp.int32)
        ib = fb.astype(jnp.int32)
        lvl_off = np.int32(2 * _PLANE_T * q)
        out = []
        if dense:
            base = ia + ib * np.int32(S)
            for k in range(4):
                b0, b1 = k & 1, (k >> 1) & 1
                m = base + np.int32(b0 + b1 * S)
                e0 = m + m - (m & np.int32(127)) + lvl_off
                out.append((e0, e0 + np.int32(128)))
        else:
            h1 = ib * _P1
            h1p = h1 + _P1
            iap = ia + np.int32(1)
            msk = np.int32(_PLANE_T - 1)
            for k in range(4):
                b0, b1 = k & 1, (k >> 1) & 1
                t = (iap if b0 else ia) ^ (h1p if b1 else h1)
                m = t & msk
                e0 = m + m - (m & np.int32(127)) + lvl_off
                out.append((e0, e0 + np.int32(128)))
        return out

    def weights(xav, xbv):
        rf = jnp.float32(res)
        fa = xav * rf
        fb = xbv * rf
        fra = fa - fa.astype(jnp.int32).astype(jnp.float32)
        frb = fb - fb.astype(jnp.int32).astype(jnp.float32)
        wa = (jnp.float32(1.0) - fra, fra)
        wb = (jnp.float32(1.0) - frb, frb)
        return [wa[k & 1] * wb[(k >> 1) & 1] for k in range(4)]

    return indices, weights


@functools.cache
def _make_sc_encode():
    mesh = plsc.VectorSubcoreMesh(
        core_axis_name="c", subcore_axis_name="s", num_cores=2, num_subcores=16)
    return pl.kernel(
        _sc_encode_body,
        out_type=jax.ShapeDtypeStruct((56, _N), jnp.float32),
        mesh=mesh,
        scratch_types=[
            pltpu.VMEM((_C,), jnp.float32),
            pltpu.VMEM((_C,), jnp.float32),
            pltpu.VMEM((_C,), jnp.float32),
            pltpu.VMEM((16 * _C,), jnp.int32),
            pltpu.VMEM((16 * _C,), jnp.int32),
            pltpu.VMEM((16 * _C,), jnp.float32),
            pltpu.VMEM((16 * _C,), jnp.float32),
            pltpu.VMEM((56, _C), jnp.float32),
            pltpu.SemaphoreType.DMA,
            pltpu.SemaphoreType.DMA,
        ],
    )


def _sc_encode_body(x0_hbm, x1_hbm, x2_hbm, fg_hbm, fp0_hbm, fp1_hbm, fp2_hbm,
                    enc_hbm, xa_v, xb_v, xc_v, idx0_v, idx1_v, dst0_v, dst1_v,
                    acc_v, sem0, sem1):
    wid = lax.axis_index("s") * 2 + lax.axis_index("c")
    idx_bufs = (idx0_v, idx1_v)
    dst_bufs = (dst0_v, dst1_v)
    sems = (sem0, sem1)

    def chunk(g, _):
        base = wid * _PPW + g * _C
        pltpu.sync_copy(x0_hbm.at[pl.ds(base, _C)], xa_v)
        pltpu.sync_copy(x1_hbm.at[pl.ds(base, _C)], xb_v)
        pltpu.sync_copy(x2_hbm.at[pl.ds(base, _C)], xc_v)

        stages = []
        for l in range(_GRID_LEVELS):
            idx_fn, w_fn = _emit_grid_level(l, None, None, None, _GRID_RES[l])
            stages.append((2 * l, (xa_v, xb_v, xc_v),
                           (lambda *xs, f=idx_fn: f(None, *xs)), w_fn, 8,
                           fg_hbm))
        plane_coords = [(xa_v, xb_v), (xb_v, xc_v), (xc_v, xa_v)]
        plane_tbls = [fp0_hbm, fp1_hbm, fp2_hbm]
        for p in range(3):
            for q in range(4):
                idx_fn, w_fn = _emit_plane_level(p, q)
                stages.append((32 + (p * 4 + q) * 2, plane_coords[p],
                               idx_fn, w_fn, 4, plane_tbls[p]))

        def pass1(st, b):
            row0, xrefs, idx_fn, w_fn, ncorner, tbl = st
            idx_v = idx_bufs[b]

            def p1(step, _):
                j0 = step * 16
                xs = [r[pl.ds(j0, 16)] for r in xrefs]
                pairs = idx_fn(*xs)
                for k, (e0, e1) in enumerate(pairs):
                    idx_v[pl.ds((2 * k) * _C + j0, 16)] = e0
                    idx_v[pl.ds((2 * k + 1) * _C + j0, 16)] = e1
                return ()

            lax.fori_loop(0, _STEPS, p1, ())

        def fire(st, b):
            row0, xrefs, idx_fn, w_fn, ncorner, tbl = st
            nidx = 2 * ncorner * _C
            sub = min(_SUB, nidx)
            descs = []
            for u in range(nidx // sub):
                descs.append(pltpu.async_copy(
                    tbl.at[idx_bufs[b].at[pl.ds(u * sub, sub)]],
                    dst_bufs[b].at[pl.ds(u * sub, sub)], sems[b]))
            return descs

        def pass2(st, b):
            row0, xrefs, idx_fn, w_fn, ncorner, tbl = st
            dst_v = dst_bufs[b]

            def p2(step, _):
                j0 = step * 16
                xs = [r[pl.ds(j0, 16)] for r in xrefs]
                ws = w_fn(*xs)
                acc0 = None
                acc1 = None
                for k in range(ncorner):
                    f0 = dst_v[pl.ds((2 * k) * _C + j0, 16)]
                    f1 = dst_v[pl.ds((2 * k + 1) * _C + j0, 16)]
                    w = ws[k]
                    if acc0 is None:
                        acc0 = w * f0
                        acc1 = w * f1
                    else:
                        acc0 = acc0 + w * f0
                        acc1 = acc1 + w * f1
                acc_v[row0, pl.ds(j0, 16)] = acc0
                acc_v[row0 + 1, pl.ds(j0, 16)] = acc1
                return ()

            lax.fori_loop(0, _STEPS, p2, ())

        pending = None
        for i, st in enumerate(stages):
            b = i % 2
            pass1(st, b)
            d = fire(st, b)
            if pending is not None:
                for dd in pending:
                    dd.wait()
                pass2(stages[i - 1], 1 - b)
            pending = d
        for dd in pending:
            dd.wait()
        pass2(stages[-1], (len(stages) - 1) % 2)

        pltpu.sync_copy(acc_v, enc_hbm.at[:, pl.ds(base, _C)])
        return ()

    lax.fori_loop(0, _NCH, chunk, ())


def _mlp_body(enc_ref, zt_ref, w0a_ref, w0b_ref, w1_ref, w2_ref, out_ref):
    pe = jnp.sin(zt_ref[...])
    h = jnp.dot(w0a_ref[...], enc_ref[...], preferred_element_type=jnp.float32)
    h = h + jnp.dot(w0b_ref[...], pe, preferred_element_type=jnp.float32)
    h = jnp.maximum(h, 0.0)
    h = jnp.dot(w1_ref[...], h, preferred_element_type=jnp.float32)
    h = jnp.maximum(h, 0.0)
    out_ref[...] = jnp.dot(w2_ref[...], h, preferred_element_type=jnp.float32)


def _mlp(encT, zt, W0aT, W0bT, W1T, W2T):
    n = encT.shape[1]
    B = 2048
    return pl.pallas_call(
        _mlp_body,
        grid=(n // B,),
        in_specs=[
            pl.BlockSpec((56, B), lambda i: (0, i)),
            pl.BlockSpec((24, B), lambda i: (0, i)),
            pl.BlockSpec((64, 56), lambda i: (0, 0)),
            pl.BlockSpec((64, 24), lambda i: (0, 0)),
            pl.BlockSpec((64, 64), lambda i: (0, 0)),
            pl.BlockSpec((16, 64), lambda i: (0, 0)),
        ],
        out_specs=pl.BlockSpec((16, B), lambda i: (0, i)),
        out_shape=jax.ShapeDtypeStruct((16, n), jnp.float32),
    )(encT, zt, W0aT, W0bT, W1T, W2T)


def kernel(in_tensor, grid_table, plane0, plane1, plane2, W0, W1, W2):
    x0 = in_tensor[:, 0]
    x1 = in_tensor[:, 1]
    x2 = in_tensor[:, 2]
    def tiled_flat(tbl):
        lv, t, _ = tbl.shape
        return tbl.reshape(lv, t // 128, 128, 2).transpose(0, 1, 3, 2).reshape(-1)

    fg = tiled_flat(grid_table)
    fp0 = tiled_flat(plane0)
    fp1 = tiled_flat(plane1)
    fp2 = tiled_flat(plane2)
    encT = _make_sc_encode()(x0, x1, x2, fg, fp0, fp1, fp2)

    xT = in_tensor.T
    tiled = jnp.tile(xT, (4, 1))
    scales = jnp.repeat(jnp.asarray([1.0, 2.0, 4.0, 8.0], jnp.float32), 3)[:, None]
    z12 = tiled * scales
    zt = jnp.concatenate([z12, z12 + jnp.float32(0.5 * math.pi)], axis=0)

    outT = _mlp(encT, zt, W0[:56].T, W0[56:].T, W1.T, W2.T)
    return outT.T


# speedup vs baseline: 1.6582x; 1.6582x over previous
import functools
import math

import jax
import jax.numpy as jnp
import numpy as np
from jax import lax
from jax.experimental import pallas as pl
from jax.experimental.pallas import tpu as pltpu
import jax.experimental.pallas.tpu_sc as plsc

_N = 524288
_GRID_LEVELS = 16
_GRID_LOG2_T = 19
_GRID_BASE = 16
_MAX_RES = 1024
_GRID_PLS = float(np.exp((np.log(_MAX_RES) - np.log(_GRID_BASE)) / (_GRID_LEVELS - 1)))
_PLANE_LOG2_T = 17
_PLANE_BASE = _MAX_RES // 4
_POS_DEG = 4
_P1 = np.int32(np.uint32(2654435761).astype(np.int32))
_P2 = np.int32(805459861)

_NW = 32
_PPW = _N // _NW
_C = 256
_STEPS = _C // 16
_NCH = _PPW // _C
_SUB = 2048

_E_GRID = _GRID_LEVELS * (1 << _GRID_LOG2_T) * 2
_E_PLANE = 4 * (1 << _PLANE_LOG2_T) * 2
_E_TOT = _E_GRID + 3 * _E_PLANE
_ZCH = 8192
_ROWS16 = _E_TOT // 16
_GRID_ROW_OFF = 0
_PLANE_ROW_OFF = _E_GRID // 16
_PLANE_ROWS = _E_PLANE // 4 // 16

_GRID_RES = [int(np.floor(_GRID_BASE * (_GRID_PLS ** l))) for l in range(_GRID_LEVELS)]
_GRID_T = 1 << _GRID_LOG2_T
_PLANE_T = 1 << _PLANE_LOG2_T
_PLANE_RES = [_PLANE_BASE * (2 ** q) for q in range(4)]
_FLAT_GRID_OFF = 0
_FLAT_PLANE_OFF = _GRID_LEVELS * _GRID_T


def _emit_grid_level(l, xa, xb, xc, res):
    S = res + 1
    dense = S ** 3 <= _GRID_T
    row_off = l * _GRID_T

    def indices(j0, xav, xbv, xcv):
        rf = jnp.float32(res)
        fa = xav * rf
        fb = xbv * rf
        fc = xcv * rf
        ia = fa.astype(jnp.int32)
        ib = fb.astype(jnp.int32)
        ic = fc.astype(jnp.int32)
        row_off = np.int32((2 * _GRID_T // 16) * l)
        out = []
        if dense:
            base = ia + ib * np.int32(S) + ic * np.int32(S * S)
            for k in range(8):
                b0, b1, b2 = k & 1, (k >> 1) & 1, (k >> 2) & 1
                m = base + np.int32(b0 + b1 * S + b2 * S * S)
                gidx = lax.shift_right_logical(m, 3) + row_off
                s2 = (m & np.int32(7)) * np.int32(2)
                out.append((gidx, s2))
        else:
            h1 = ib * _P1
            h1p = h1 + _P1
            h2 = ic * _P2
            h2p = h2 + _P2
            iap = ia + np.int32(1)
            msk = np.int32(_GRID_T - 1)
            for k in range(8):
                b0, b1, b2 = k & 1, (k >> 1) & 1, (k >> 2) & 1
                t = (iap if b0 else ia) ^ (h1p if b1 else h1) ^ (h2p if b2 else h2)
                m = t & msk
                gidx = lax.shift_right_logical(m, 3) + row_off
                s2 = (m & np.int32(7)) * np.int32(2)
                out.append((gidx, s2))
        return out

    def weights(xav, xbv, xcv):
        rf = jnp.float32(res)
        fa = xav * rf
        fb = xbv * rf
        fc = xcv * rf
        fra = fa - fa.astype(jnp.int32).astype(jnp.float32)
        frb = fb - fb.astype(jnp.int32).astype(jnp.float32)
        frc = fc - fc.astype(jnp.int32).astype(jnp.float32)
        wa = (jnp.float32(1.0) - fra, fra)
        wb = (jnp.float32(1.0) - frb, frb)
        wc = (jnp.float32(1.0) - frc, frc)
        ws = []
        for k in range(8):
            b0, b1, b2 = k & 1, (k >> 1) & 1, (k >> 2) & 1
            ws.append(wa[b0] * wb[b1] * wc[b2])
        return ws

    return indices, weights


def _emit_plane_level(p, q):
    res = _PLANE_RES[q]
    S = res + 1
    dense = S * S <= _PLANE_T
    row_off = _FLAT_PLANE_OFF + (p * 4 + q) * _PLANE_T

    def indices(xav, xbv):
        rf = jnp.float32(res)
        fa = xav * rf
        fb = xbv * rf
        ia = fa.astype(jnp.int32)
        ib = fb.astype(jnp.int32)
        row_off = np.int32(_PLANE_ROW_OFF + p * (_E_PLANE // 16)
                           + q * _PLANE_ROWS)
        out = []
        if dense:
            base = ia + ib * np.int32(S)
            for k in range(4):
                b0, b1 = k & 1, (k >> 1) & 1
                m = base + np.int32(b0 + b1 * S)
                gidx = lax.shift_right_logical(m, 3) + row_off
                s2 = (m & np.int32(7)) * np.int32(2)
                out.append((gidx, s2))
        else:
            h1 = ib * _P1
            h1p = h1 + _P1
            iap = ia + np.int32(1)
            msk = np.int32(_PLANE_T - 1)
            for k in range(4):
                b0, b1 = k & 1, (k >> 1) & 1
                t = (iap if b0 else ia) ^ (h1p if b1 else h1)
                m = t & msk
                gidx = lax.shift_right_logical(m, 3) + row_off
                s2 = (m & np.int32(7)) * np.int32(2)
                out.append((gidx, s2))
        return out

    def weights(xav, xbv):
        rf = jnp.float32(res)
        fa = xav * rf
        fb = xbv * rf
        fra = fa - fa.astype(jnp.int32).astype(jnp.float32)
        frb = fb - fb.astype(jnp.int32).astype(jnp.float32)
        wa = (jnp.float32(1.0) - fra, fra)
        wb = (jnp.float32(1.0) - frb, frb)
        return [wa[k & 1] * wb[(k >> 1) & 1] for k in range(4)]

    return indices, weights


def _sc_mesh():
    return plsc.VectorSubcoreMesh(
        core_axis_name="c", subcore_axis_name="s", num_cores=2, num_subcores=16)


@functools.cache
def _make_zip():
    return pl.kernel(
        _zip_body,
        out_type=jax.ShapeDtypeStruct((_E_TOT,), jnp.float32),
        mesh=_sc_mesh(),
        compiler_params=pltpu.CompilerParams(needs_layout_passes=False),
        scratch_types=[
            pltpu.VMEM((_ZCH,), jnp.float32),
            pltpu.VMEM((_ZCH,), jnp.float32),
        ],
    )


def _zip_body(fg_hbm, fp0_hbm, fp1_hbm, fp2_hbm, out_hbm, in_v, out_v):
    wid = lax.axis_index("s") * 2 + lax.axis_index("c")
    iot2 = lax.iota(jnp.int32, 16) * np.int32(2)

    def table(tbl_ref, e_base, e_total):
        per_w = e_total // _NW
        start = wid * per_w

        def ch(c, _):
            off = start + c * _ZCH
            pltpu.sync_copy(tbl_ref.at[pl.ds(off, _ZCH)], in_v)

            def blk(b, _):
                boff = b * 256
                for u in range(8):
                    a = in_v[pl.ds(boff + u * 16, 16)]
                    bb = in_v[pl.ds(boff + 128 + u * 16, 16)]
                    idx = iot2 + (boff + np.int32(u * 32))
                    plsc.store_scatter(out_v, [idx], a)
                    plsc.store_scatter(out_v, [idx + np.int32(1)], bb)
                return ()

            lax.fori_loop(0, _ZCH // 256, blk, ())
            pltpu.sync_copy(out_v, out_hbm.at[pl.ds(e_base + off, _ZCH)])
            return ()

        lax.fori_loop(0, per_w // _ZCH, ch, ())

    table(fg_hbm, 0, _E_GRID)
    table(fp0_hbm, _E_GRID, _E_PLANE)
    table(fp1_hbm, _E_GRID + _E_PLANE, _E_PLANE)
    table(fp2_hbm, _E_GRID + 2 * _E_PLANE, _E_PLANE)


@functools.cache
def _make_sc_encode():
    return pl.kernel(
        _sc_encode_body,
        out_type=jax.ShapeDtypeStruct((_N // 128, 56, 128), jnp.float32),
        mesh=_sc_mesh(),
        compiler_params=pltpu.CompilerParams(
            use_tc_tiling_on_sc=False, needs_layout_passes=False),
        scratch_types=[
            pltpu.VMEM((_C,), jnp.float32),
            pltpu.VMEM((_C,), jnp.float32),
            pltpu.VMEM((_C,), jnp.float32),
            pltpu.VMEM((8 * _C,), jnp.int32),
            pltpu.VMEM((8 * _C,), jnp.int32),
            pltpu.VMEM((8 * _C,), jnp.int32),
            pltpu.VMEM((8 * _C,), jnp.int32),
            pltpu.VMEM((8 * _C, 16), jnp.float32),
            pltpu.VMEM((8 * _C, 16), jnp.float32),
            pltpu.VMEM((56, _C), jnp.float32),
            pltpu.SemaphoreType.DMA,
            pltpu.SemaphoreType.DMA,
        ],
    )


def _sc_encode_body(x0_hbm, x1_hbm, x2_hbm, tbl_hbm, enc_hbm,
                    xa_v, xb_v, xc_v, idx0_v, idx1_v, sub0_v, sub1_v,
                    dst0_v, dst1_v, acc_v, sem0, sem1):
    wid = lax.axis_index("s") * 2 + lax.axis_index("c")
    idx_bufs = (idx0_v, idx1_v)
    sub_bufs = (sub0_v, sub1_v)
    dst_bufs = (dst0_v, dst1_v)
    sems = (sem0, sem1)
    iot = lax.iota(jnp.int32, 16)

    def chunk(g, _):
        base = wid * _PPW + g * _C
        pltpu.sync_copy(x0_hbm.at[pl.ds(base, _C)], xa_v)
        pltpu.sync_copy(x1_hbm.at[pl.ds(base, _C)], xb_v)
        pltpu.sync_copy(x2_hbm.at[pl.ds(base, _C)], xc_v)

        stages = []
        for l in range(_GRID_LEVELS):
            idx_fn, w_fn = _emit_grid_level(l, None, None, None, _GRID_RES[l])
            stages.append((2 * l, (xa_v, xb_v, xc_v),
                           (lambda *xs, f=idx_fn: f(None, *xs)), w_fn, 8))
        plane_coords = [(xa_v, xb_v), (xb_v, xc_v), (xc_v, xa_v)]
        for p in range(3):
            for q in range(4):
                idx_fn, w_fn = _emit_plane_level(p, q)
                stages.append((32 + (p * 4 + q) * 2, plane_coords[p],
                               idx_fn, w_fn, 4))

        def pass1(st, b):
            row0, xrefs, idx_fn, w_fn, ncorner = st
            idx_v = idx_bufs[b]
            sub_v = sub_bufs[b]

            def p1(step, _):
                j0 = step * 16
                xs = [r[pl.ds(j0, 16)] for r in xrefs]
                pairs = idx_fn(*xs)
                for k, (gidx, s2) in enumerate(pairs):
                    idx_v[pl.ds(k * _C + j0, 16)] = gidx
                    sub_v[pl.ds(k * _C + j0, 16)] = s2
                return ()

            lax.fori_loop(0, _STEPS, p1, ())

        def fire(st, b):
            row0, xrefs, idx_fn, w_fn, ncorner = st
            nidx = ncorner * _C
            sub = min(_SUB, nidx)
            descs = []
            for u in range(nidx // sub):
                descs.append(pltpu.async_copy(
                    tbl_hbm.at[idx_bufs[b].at[pl.ds(u * sub, sub)]],
                    dst_bufs[b].at[pl.ds(u * sub, sub)], sems[b]))
            return descs

        def pass2(st, b):
            row0, xrefs, idx_fn, w_fn, ncorner = st
            dst_v = dst_bufs[b]
            sub_v = sub_bufs[b]

            def p2(step, _):
                j0 = step * 16
                xs = [r[pl.ds(j0, 16)] for r in xrefs]
                ws = w_fn(*xs)
                rb = iot + j0
                acc0 = None
                acc1 = None
                for k in range(ncorner):
                    s2 = sub_v[pl.ds(k * _C + j0, 16)]
                    rows = rb + np.int32(k * _C)
                    f0 = plsc.load_gather(dst_v, [rows, s2])
                    f1 = plsc.load_gather(dst_v, [rows, s2 + np.int32(1)])
                    w = ws[k]
                    if acc0 is None:
                        acc0 = w * f0
                        acc1 = w * f1
                    else:
                        acc0 = acc0 + w * f0
                        acc1 = acc1 + w * f1
                acc_v[row0, pl.ds(j0, 16)] = acc0
                acc_v[row0 + 1, pl.ds(j0, 16)] = acc1
                return ()

            lax.fori_loop(0, _STEPS, p2, ())

        pending = None
        for i, st in enumerate(stages):
            b = i % 2
            pass1(st, b)
            d = fire(st, b)
            if pending is not None:
                for dd in pending:
                    dd.wait()
                pass2(stages[i - 1], 1 - b)
            pending = d
        for dd in pending:
            dd.wait()
        pass2(stages[-1], (len(stages) - 1) % 2)

        blk0 = wid * (_PPW // 128) + g * (_C // 128)
        for h in range(_C // 128):
            pltpu.sync_copy(acc_v.at[:, pl.ds(h * 128, 128)],
                            enc_hbm.at[blk0 + h])
        return ()

    lax.fori_loop(0, _NCH, chunk, ())


def _mlp_body(enc_ref, zt_ref, w0a_ref, w0b_ref, w1_ref, w2_ref, out_ref):
    nb = enc_ref.shape[0]
    for j in range(nb):
        e = enc_ref[j]
        pe = jnp.sin(zt_ref[:, j * 128:(j + 1) * 128])
        h = jnp.dot(w0a_ref[...], e, preferred_element_type=jnp.float32)
        h = h + jnp.dot(w0b_ref[...], pe, preferred_element_type=jnp.float32)
        h = jnp.maximum(h, 0.0)
        h = jnp.dot(w1_ref[...], h, preferred_element_type=jnp.float32)
        h = jnp.maximum(h, 0.0)
        out_ref[:, j * 128:(j + 1) * 128] = jnp.dot(
            w2_ref[...], h, preferred_element_type=jnp.float32)


def _mlp(enc3, zt, W0aT, W0bT, W1T, W2T):
    n = zt.shape[1]
    B = 2048
    nb = B // 128
    return pl.pallas_call(
        _mlp_body,
        grid=(n // B,),
        in_specs=[
            pl.BlockSpec((nb, 56, 128), lambda i: (i, 0, 0)),
            pl.BlockSpec((24, B), lambda i: (0, i)),
            pl.BlockSpec((64, 56), lambda i: (0, 0)),
            pl.BlockSpec((64, 24), lambda i: (0, 0)),
            pl.BlockSpec((64, 64), lambda i: (0, 0)),
            pl.BlockSpec((16, 64), lambda i: (0, 0)),
        ],
        out_specs=pl.BlockSpec((16, B), lambda i: (0, i)),
        out_shape=jax.ShapeDtypeStruct((16, n), jnp.float32),
    )(enc3, zt, W0aT, W0bT, W1T, W2T)


def kernel(in_tensor, grid_table, plane0, plane1, plane2, W0, W1, W2):
    x0 = in_tensor[:, 0]
    x1 = in_tensor[:, 1]
    x2 = in_tensor[:, 2]
    def tiled_flat(tbl):
        lv, t, _ = tbl.shape
        return tbl.reshape(lv, t // 128, 128, 2).transpose(0, 1, 3, 2).reshape(-1)

    fg = tiled_flat(grid_table)
    fp0 = tiled_flat(plane0)
    fp1 = tiled_flat(plane1)
    fp2 = tiled_flat(plane2)
    flat2 = _make_zip()(fg, fp0, fp1, fp2)
    tbl16 = flat2.reshape(_ROWS16, 16)
    enc3 = _make_sc_encode()(x0, x1, x2, tbl16)

    xT = in_tensor.T
    tiled = jnp.tile(xT, (4, 1))
    scales = jnp.repeat(jnp.asarray([1.0, 2.0, 4.0, 8.0], jnp.float32), 3)[:, None]
    z12 = tiled * scales
    zt = jnp.concatenate([z12, z12 + jnp.float32(0.5 * math.pi)], axis=0)

    outT = _mlp(enc3, zt, W0[:56].T, W0[56:].T, W1.T, W2.T)
    return outT.T
